# baseline (speedup 1.0000x reference)
import functools

import jax
import jax.numpy as jnp
from jax import lax
from jax.experimental import pallas as pl
from jax.experimental.pallas import tpu as pltpu
from jax.experimental.pallas import tpu_sc as plsc

D_IN = 4096
H1 = 200
H = 50
D_OUT = 4096
DA = 20
BB = 16
LL = 1024
NLANE = 16
NPLANE = 4

TILE = 512
NTILES = D_IN // TILE


def _sc_counts(idx, zeros_flat):
  mesh = plsc.VectorSubcoreMesh(core_axis_name="c", subcore_axis_name="s")

  @functools.partial(
      pl.kernel,
      mesh=mesh,
      out_type=jax.ShapeDtypeStruct((BB * NPLANE, D_IN), jnp.float32),
      compiler_params=pltpu.CompilerParams(needs_layout_passes=False),
      scratch_types=[
          pltpu.VMEM((LL,), jnp.int32),
          pltpu.VMEM((NPLANE, D_IN), jnp.float32),
          pltpu.SemaphoreType.DMA,
          pltpu.SemaphoreType.DMA,
      ],
  )
  def hist(idx_hbm, zeros_hbm, out_hbm, idx_v, hist_v, sem1, sem2):
    c = lax.axis_index("c")
    s = lax.axis_index("s")
    wid = s * 2 + c

    @pl.when(wid < BB)
    def _():
      cp1 = pltpu.async_copy(idx_hbm.at[wid], idx_v, sem1)
      cp2 = pltpu.async_copy(zeros_hbm, hist_v, sem2)
      cp1.wait()
      cp2.wait()

      lane = lax.iota(jnp.int32, NLANE)
      ones16 = jnp.ones((NLANE,), jnp.float32)
      planes = [(lane - g * NPLANE) & (NPLANE - 1)
                for g in range(NLANE // NPLANE)]
      masks = [(lane >= g * NPLANE) & (lane < (g + 1) * NPLANE)
               for g in range(NLANE // NPLANE)]

      def scat_body(j, _):
        iv = idx_v[pl.ds(j * NLANE, NLANE)]
        for g in range(NLANE // NPLANE):
          plsc.addupdate_scatter(hist_v, [planes[g], iv], ones16,
                                 mask=masks[g])
        return ()

      lax.fori_loop(0, LL // NLANE, scat_body, ())

      pltpu.sync_copy(hist_v, out_hbm.at[pl.ds(wid * NPLANE, NPLANE)])

  return hist(idx, zeros_flat)


def _nt(a, b):
  return lax.dot_general(a, b, (((1,), (1,)), ((), ())),
                         preferred_element_type=jnp.float32)


def _nn(a, b):
  return lax.dot_general(a, b, (((1,), (0,)), ((), ())),
                         preferred_element_type=jnp.float32)


def _tn(a, b):
  return lax.dot_general(a, b, (((0,), (1,)), ((), ())),
                         preferred_element_type=jnp.float32)


def _onehot(rows, cols, fn, div=1):
  r = lax.broadcasted_iota(jnp.int32, (rows, cols), 0)
  c = lax.broadcasted_iota(jnp.int32, (rows, cols), 1)
  return jnp.where(fn(r) == c // div, 1.0, 0.0).astype(jnp.float32)


def _main_body(cnt_ref, w1t_ref, w1f_ref, pc_ref, w4_ref, a_ref, wsa_ref,
               bsa_ref, w2_ref, b2_ref, w3_ref, b3_ref, b4_ref, out_ref,
               acc_ref, cf_ref, ef_ref):
  t = pl.program_id(0)

  c3 = cnt_ref[...]
  fold = _onehot(BB, BB * NPLANE, lambda r: r, div=NPLANE)
  c_t = _nn(fold, c3)
  w1 = w1t_ref[...]

  s_t = jnp.tanh(_nn(a_ref[...], w1))
  e_t = jnp.exp(s_t)

  @pl.when(t == 0)
  def _():
    acc_ref[...] = jnp.zeros_like(acc_ref)

  cf_ref[:, pl.ds(t * TILE, TILE)] = c_t
  ef_ref[:, pl.ds(t * TILE, TILE)] = e_t

  gt = _tn(w1, w4_ref[...])
  q = gt * pc_ref[...]
  acc_ref[...] += _nn(c_t, q)

  @pl.when(t == NTILES - 1)
  def _():
    cf = cf_ref[...]
    ef = ef_ref[...]
    denom = _nt(cf, ef)
    ohb = _onehot(BB * DA, BB, lambda r: r // DA)
    oha = _onehot(BB * DA, DA, lambda r: r % DA)
    r_full = _nn(ohb, cf) * _nn(oha, ef)
    numer = _nt(r_full, w1f_ref[...]).reshape(BB, DA, H1)
    emb = numer / denom[:, :, None]
    lz = jnp.sum(emb * wsa_ref[...][0][None, :, None], axis=1) + bsa_ref[0]
    z = jnp.tanh(lz)
    z = jnp.tanh(_nt(z, w2_ref[...]) + b2_ref[...][None, :])
    dz = jnp.tanh(_nt(z, w3_ref[...]) + b3_ref[...][None, :])
    y = _nt(dz, w4_ref[...]) + b4_ref[...][None, :] + acc_ref[...]
    out_ref[...] = jax.nn.sigmoid(y)


def _tc_main(counts2, w1t, pc, w4, a, wsa, bsa, w2, b2, w3, b3, b4):
  grid = (NTILES,)
  full = lambda shape: pl.BlockSpec(shape, lambda t: (0,) * len(shape))
  return pl.pallas_call(
      _main_body,
      grid=grid,
      in_specs=[
          pl.BlockSpec((BB * NPLANE, TILE), lambda t: (0, t)),
          pl.BlockSpec((H1, TILE), lambda t: (0, t)),
          full((H1, D_IN)),
          pl.BlockSpec((TILE, D_OUT), lambda t: (t, 0)),
          full((D_OUT, H1)),
          full((DA, H1)),
          full((1, DA)),
          full((1,)),
          full((H, H1)),
          full((H,)),
          full((H1, H)),
          full((H1,)),
          full((D_OUT,)),
      ],
      out_specs=pl.BlockSpec((BB, D_OUT), lambda t: (0, 0)),
      out_shape=jax.ShapeDtypeStruct((BB, D_OUT), jnp.float32),
      scratch_shapes=[
          pltpu.VMEM((BB, D_OUT), jnp.float32),
          pltpu.VMEM((BB, D_IN), jnp.float32),
          pltpu.VMEM((DA, D_IN), jnp.float32),
      ],
  )(counts2, w1t, w1t, pc, w4, a, wsa, bsa, w2, b2, w3, b3, b4)


def kernel(batch_item_index, place_correlation, W1, W2, b2, W3, b3, W4, b4,
           A, Wsa, bsa):
  zeros_flat = jnp.zeros((NPLANE, D_IN), jnp.float32)
  counts2 = _sc_counts(batch_item_index, zeros_flat)
  return _tc_main(
      counts2,
      W1,
      place_correlation,
      W4,
      A,
      Wsa,
      bsa,
      W2,
      b2,
      W3,
      b3,
      b4,
  )


# speedup vs baseline: 1.0578x; 1.0578x over previous
import functools

import jax
import jax.numpy as jnp
from jax import lax
from jax.experimental import pallas as pl
from jax.experimental.pallas import tpu as pltpu
from jax.experimental.pallas import tpu_sc as plsc

D_IN = 4096
H1 = 200
H = 50
D_OUT = 4096
DA = 20
BB = 16
LL = 1024
NLANE = 16
NPLANE = 4

TILE = 512
NTILES = D_IN // TILE


def _sc_counts(idx, zeros_flat):
  mesh = plsc.VectorSubcoreMesh(core_axis_name="c", subcore_axis_name="s")

  @functools.partial(
      pl.kernel,
      mesh=mesh,
      out_type=jax.ShapeDtypeStruct((BB * NPLANE, D_IN), jnp.float32),
      compiler_params=pltpu.CompilerParams(needs_layout_passes=False),
      scratch_types=[
          pltpu.VMEM((LL,), jnp.int32),
          pltpu.VMEM((NPLANE, D_IN), jnp.float32),
          pltpu.SemaphoreType.DMA,
          pltpu.SemaphoreType.DMA,
      ],
  )
  def hist(idx_hbm, zeros_hbm, out_hbm, idx_v, hist_v, sem1, sem2):
    c = lax.axis_index("c")
    s = lax.axis_index("s")
    wid = s * 2 + c

    @pl.when(wid < BB)
    def _():
      cp1 = pltpu.async_copy(idx_hbm.at[wid], idx_v, sem1)
      cp2 = pltpu.async_copy(zeros_hbm, hist_v, sem2)
      cp1.wait()
      cp2.wait()

      lane = lax.iota(jnp.int32, NLANE)
      ones16 = jnp.ones((NLANE,), jnp.float32)
      planes = [(lane - g * NPLANE) & (NPLANE - 1)
                for g in range(NLANE // NPLANE)]
      masks = [(lane >= g * NPLANE) & (lane < (g + 1) * NPLANE)
               for g in range(NLANE // NPLANE)]

      def scat_body(j, _):
        iv = idx_v[pl.ds(j * NLANE, NLANE)]
        for g in range(NLANE // NPLANE):
          plsc.addupdate_scatter(hist_v, [planes[g], iv], ones16,
                                 mask=masks[g])
        return ()

      lax.fori_loop(0, LL // NLANE, scat_body, ())

      pltpu.sync_copy(hist_v, out_hbm.at[pl.ds(wid * NPLANE, NPLANE)])

  return hist(idx, zeros_flat)


def _nt(a, b):
  return lax.dot_general(a, b, (((1,), (1,)), ((), ())),
                         preferred_element_type=jnp.float32)


def _nn(a, b):
  return lax.dot_general(a, b, (((1,), (0,)), ((), ())),
                         preferred_element_type=jnp.float32)


def _tn(a, b):
  return lax.dot_general(a, b, (((0,), (1,)), ((), ())),
                         preferred_element_type=jnp.float32)


def _onehot(rows, cols, fn, div=1):
  r = lax.broadcasted_iota(jnp.int32, (rows, cols), 0)
  c = lax.broadcasted_iota(jnp.int32, (rows, cols), 1)
  return jnp.where(fn(r) == c // div, 1.0, 0.0).astype(jnp.float32)


def _main_body(cnt_ref, w1t_ref, pc_ref, w4_ref, a_ref, wsa_ref, bsa_ref,
               w2_ref, b2_ref, w3_ref, b3_ref, b4_ref, out_ref,
               acc_ref, numer_ref, denom_ref):
  t = pl.program_id(0)

  c3 = cnt_ref[...]
  fold = _onehot(BB, BB * NPLANE, lambda r: r, div=NPLANE)
  c_t = _nn(fold, c3)
  w1 = w1t_ref[...]

  s_t = jnp.tanh(_nn(a_ref[...], w1))
  e_t = jnp.exp(s_t)

  @pl.when(t == 0)
  def _():
    acc_ref[...] = jnp.zeros_like(acc_ref)
    numer_ref[...] = jnp.zeros_like(numer_ref)
    denom_ref[...] = jnp.zeros_like(denom_ref)

  denom_ref[...] += _nt(c_t, e_t)
  ohb = _onehot(BB * DA, BB, lambda r: r // DA)
  oha = _onehot(BB * DA, DA, lambda r: r % DA)
  r_t = _nn(ohb, c_t) * _nn(oha, e_t)
  numer_ref[...] += _nt(r_t, w1)

  gt = _tn(w1, w4_ref[...])
  q = gt * pc_ref[...]
  acc_ref[...] += _nn(c_t, q)

  @pl.when(t == NTILES - 1)
  def _():
    numer = numer_ref[...].reshape(BB, DA, H1)
    denom = denom_ref[...]
    emb = numer / denom[:, :, None]
    lz = jnp.sum(emb * wsa_ref[...][0][None, :, None], axis=1) + bsa_ref[0]
    z = jnp.tanh(lz)
    z = jnp.tanh(_nt(z, w2_ref[...]) + b2_ref[...][None, :])
    dz = jnp.tanh(_nt(z, w3_ref[...]) + b3_ref[...][None, :])
    y = _nt(dz, w4_ref[...]) + b4_ref[...][None, :] + acc_ref[...]
    out_ref[...] = jax.nn.sigmoid(y)


def _tc_main(counts2, w1t, pc, w4, a, wsa, bsa, w2, b2, w3, b3, b4):
  grid = (NTILES,)
  full = lambda shape: pl.BlockSpec(shape, lambda t: (0,) * len(shape))
  return pl.pallas_call(
      _main_body,
      grid=grid,
      in_specs=[
          pl.BlockSpec((BB * NPLANE, TILE), lambda t: (0, t)),
          pl.BlockSpec((H1, TILE), lambda t: (0, t)),
          pl.BlockSpec((TILE, D_OUT), lambda t: (t, 0)),
          full((D_OUT, H1)),
          full((DA, H1)),
          full((1, DA)),
          full((1,)),
          full((H, H1)),
          full((H,)),
          full((H1, H)),
          full((H1,)),
          full((D_OUT,)),
      ],
      out_specs=pl.BlockSpec((BB, D_OUT), lambda t: (0, 0)),
      out_shape=jax.ShapeDtypeStruct((BB, D_OUT), jnp.float32),
      scratch_shapes=[
          pltpu.VMEM((BB, D_OUT), jnp.float32),
          pltpu.VMEM((BB * DA, H1), jnp.float32),
          pltpu.VMEM((BB, DA), jnp.float32),
      ],
  )(counts2, w1t, pc, w4, a, wsa, bsa, w2, b2, w3, b3, b4)


def kernel(batch_item_index, place_correlation, W1, W2, b2, W3, b3, W4, b4,
           A, Wsa, bsa):
  zeros_flat = jnp.zeros((NPLANE, D_IN), jnp.float32)
  counts2 = _sc_counts(batch_item_index, zeros_flat)
  return _tc_main(
      counts2,
      W1,
      place_correlation,
      W4,
      A,
      Wsa,
      bsa,
      W2,
      b2,
      W3,
      b3,
      b4,
  )


# speedup vs baseline: 1.0689x; 1.0105x over previous
import functools

import jax
import jax.numpy as jnp
from jax import lax
from jax.experimental import pallas as pl
from jax.experimental.pallas import tpu as pltpu
from jax.experimental.pallas import tpu_sc as plsc

D_IN = 4096
H1 = 200
H = 50
D_OUT = 4096
DA = 20
BB = 16
LL = 1024
NLANE = 16
NPLANE = 2

TILE = 512
NTILES = D_IN // TILE


def _sc_counts(idx, zeros_flat):
  mesh = plsc.VectorSubcoreMesh(core_axis_name="c", subcore_axis_name="s")

  @functools.partial(
      pl.kernel,
      mesh=mesh,
      out_type=jax.ShapeDtypeStruct((BB * NPLANE, D_IN), jnp.float32),
      compiler_params=pltpu.CompilerParams(needs_layout_passes=False),
      scratch_types=[
          pltpu.VMEM((LL,), jnp.int32),
          pltpu.VMEM((NPLANE, D_IN), jnp.float32),
          pltpu.SemaphoreType.DMA,
          pltpu.SemaphoreType.DMA,
      ],
  )
  def hist(idx_hbm, zeros_hbm, out_hbm, idx_v, hist_v, sem1, sem2):
    c = lax.axis_index("c")
    s = lax.axis_index("s")
    wid = s * 2 + c

    @pl.when(wid < BB)
    def _():
      cp1 = pltpu.async_copy(idx_hbm.at[wid], idx_v, sem1)
      cp2 = pltpu.async_copy(zeros_hbm, hist_v, sem2)
      cp1.wait()
      cp2.wait()

      lane = lax.iota(jnp.int32, NLANE)
      ones16 = jnp.ones((NLANE,), jnp.float32)
      planes = [(lane - g * NPLANE) & (NPLANE - 1)
                for g in range(NLANE // NPLANE)]
      masks = [(lane >= g * NPLANE) & (lane < (g + 1) * NPLANE)
               for g in range(NLANE // NPLANE)]

      def scat_body(j, _):
        iv = idx_v[pl.ds(j * NLANE, NLANE)]
        for g in range(NLANE // NPLANE):
          plsc.addupdate_scatter(hist_v, [planes[g], iv], ones16,
                                 mask=masks[g])
        return ()

      lax.fori_loop(0, LL // NLANE, scat_body, ())

      pltpu.sync_copy(hist_v, out_hbm.at[pl.ds(wid * NPLANE, NPLANE)])

  return hist(idx, zeros_flat)


def _nt(a, b):
  return lax.dot_general(a, b, (((1,), (1,)), ((), ())),
                         preferred_element_type=jnp.float32)


def _nn(a, b):
  return lax.dot_general(a, b, (((1,), (0,)), ((), ())),
                         preferred_element_type=jnp.float32)


def _tn(a, b):
  return lax.dot_general(a, b, (((0,), (1,)), ((), ())),
                         preferred_element_type=jnp.float32)


def _onehot(rows, cols, fn, div=1):
  r = lax.broadcasted_iota(jnp.int32, (rows, cols), 0)
  c = lax.broadcasted_iota(jnp.int32, (rows, cols), 1)
  return jnp.where(fn(r) == c // div, 1.0, 0.0).astype(jnp.float32)


def _main_body(cnt_ref, w1t_ref, pc_ref, w4_ref, a_ref, wsa_ref, bsa_ref,
               w2_ref, b2_ref, w3_ref, b3_ref, b4_ref, out_ref,
               acc_ref, numer_ref, denom_ref):
  t = pl.program_id(0)

  c3 = cnt_ref[...]
  fold = _onehot(BB, BB * NPLANE, lambda r: r, div=NPLANE)
  c_t = _nn(fold, c3)
  w1 = w1t_ref[...]

  s_t = jnp.tanh(_nn(a_ref[...], w1))
  e_t = jnp.exp(s_t)

  @pl.when(t == 0)
  def _():
    acc_ref[...] = jnp.zeros_like(acc_ref)
    numer_ref[...] = jnp.zeros_like(numer_ref)
    denom_ref[...] = jnp.zeros_like(denom_ref)

  denom_ref[...] += _nt(c_t, e_t)
  ohb = _onehot(BB * DA, BB, lambda r: r // DA)
  oha = _onehot(BB * DA, DA, lambda r: r % DA)
  r_t = _nn(ohb, c_t) * _nn(oha, e_t)
  numer_ref[...] += _nt(r_t, w1)

  gt = _tn(w1, w4_ref[...])
  q = gt * pc_ref[...]
  acc_ref[...] += _nn(c_t, q)

  @pl.when(t == NTILES - 1)
  def _():
    numer = numer_ref[...].reshape(BB, DA, H1)
    denom = denom_ref[...]
    emb = numer / denom[:, :, None]
    lz = jnp.sum(emb * wsa_ref[...][0][None, :, None], axis=1) + bsa_ref[0]
    z = jnp.tanh(lz)
    z = jnp.tanh(_nt(z, w2_ref[...]) + b2_ref[...][None, :])
    dz = jnp.tanh(_nt(z, w3_ref[...]) + b3_ref[...][None, :])
    y = _nt(dz, w4_ref[...]) + b4_ref[...][None, :] + acc_ref[...]
    out_ref[...] = jax.nn.sigmoid(y)


def _tc_main(counts2, w1t, pc, w4, a, wsa, bsa, w2, b2, w3, b3, b4):
  grid = (NTILES,)
  full = lambda shape: pl.BlockSpec(shape, lambda t: (0,) * len(shape))
  return pl.pallas_call(
      _main_body,
      grid=grid,
      in_specs=[
          pl.BlockSpec((BB * NPLANE, TILE), lambda t: (0, t)),
          pl.BlockSpec((H1, TILE), lambda t: (0, t)),
          pl.BlockSpec((TILE, D_OUT), lambda t: (t, 0)),
          full((D_OUT, H1)),
          full((DA, H1)),
          full((1, DA)),
          full((1,)),
          full((H, H1)),
          full((H,)),
          full((H1, H)),
          full((H1,)),
          full((D_OUT,)),
      ],
      out_specs=pl.BlockSpec((BB, D_OUT), lambda t: (0, 0)),
      out_shape=jax.ShapeDtypeStruct((BB, D_OUT), jnp.float32),
      scratch_shapes=[
          pltpu.VMEM((BB, D_OUT), jnp.float32),
          pltpu.VMEM((BB * DA, H1), jnp.float32),
          pltpu.VMEM((BB, DA), jnp.float32),
      ],
  )(counts2, w1t, pc, w4, a, wsa, bsa, w2, b2, w3, b3, b4)


def kernel(batch_item_index, place_correlation, W1, W2, b2, W3, b3, W4, b4,
           A, Wsa, bsa):
  zeros_flat = jnp.zeros((NPLANE, D_IN), jnp.float32)
  counts2 = _sc_counts(batch_item_index, zeros_flat)
  return _tc_main(
      counts2,
      W1,
      place_correlation,
      W4,
      A,
      Wsa,
      bsa,
      W2,
      b2,
      W3,
      b3,
      b4,
  )


# speedup vs baseline: 1.0761x; 1.0067x over previous
import functools

import jax
import jax.numpy as jnp
from jax import lax
from jax.experimental import pallas as pl
from jax.experimental.pallas import tpu as pltpu
from jax.experimental.pallas import tpu_sc as plsc

D_IN = 4096
H1 = 200
H = 50
D_OUT = 4096
DA = 20
BB = 16
LL = 1024
NLANE = 16
NPLANE = 1

TILE = 512
NTILES = D_IN // TILE


def _sc_counts(idx, zeros_flat):
  mesh = plsc.VectorSubcoreMesh(core_axis_name="c", subcore_axis_name="s")

  @functools.partial(
      pl.kernel,
      mesh=mesh,
      out_type=jax.ShapeDtypeStruct((BB * NPLANE, D_IN), jnp.float32),
      compiler_params=pltpu.CompilerParams(needs_layout_passes=False),
      scratch_types=[
          pltpu.VMEM((LL,), jnp.int32),
          pltpu.VMEM((NPLANE, D_IN), jnp.float32),
          pltpu.SemaphoreType.DMA,
          pltpu.SemaphoreType.DMA,
      ],
  )
  def hist(idx_hbm, zeros_hbm, out_hbm, idx_v, hist_v, sem1, sem2):
    c = lax.axis_index("c")
    s = lax.axis_index("s")
    wid = s * 2 + c

    @pl.when(wid < BB)
    def _():
      cp1 = pltpu.async_copy(idx_hbm.at[wid], idx_v, sem1)
      cp2 = pltpu.async_copy(zeros_hbm, hist_v, sem2)
      cp1.wait()
      cp2.wait()

      lane = lax.iota(jnp.int32, NLANE)
      ones16 = jnp.ones((NLANE,), jnp.float32)
      planes = [(lane - g * NPLANE) & (NPLANE - 1)
                for g in range(NLANE // NPLANE)]
      masks = [(lane >= g * NPLANE) & (lane < (g + 1) * NPLANE)
               for g in range(NLANE // NPLANE)]

      def scat_body(j, _):
        iv = idx_v[pl.ds(j * NLANE, NLANE)]
        for g in range(NLANE // NPLANE):
          plsc.addupdate_scatter(hist_v, [planes[g], iv], ones16,
                                 mask=masks[g])
        return ()

      lax.fori_loop(0, LL // NLANE, scat_body, ())

      pltpu.sync_copy(hist_v, out_hbm.at[pl.ds(wid * NPLANE, NPLANE)])

  return hist(idx, zeros_flat)


def _nt(a, b):
  return lax.dot_general(a, b, (((1,), (1,)), ((), ())),
                         preferred_element_type=jnp.float32)


def _nn(a, b):
  return lax.dot_general(a, b, (((1,), (0,)), ((), ())),
                         preferred_element_type=jnp.float32)


def _tn(a, b):
  return lax.dot_general(a, b, (((0,), (1,)), ((), ())),
                         preferred_element_type=jnp.float32)


def _onehot(rows, cols, fn, div=1):
  r = lax.broadcasted_iota(jnp.int32, (rows, cols), 0)
  c = lax.broadcasted_iota(jnp.int32, (rows, cols), 1)
  return jnp.where(fn(r) == c // div, 1.0, 0.0).astype(jnp.float32)


def _main_body(cnt_ref, w1t_ref, pc_ref, w4_ref, a_ref, wsa_ref, bsa_ref,
               w2_ref, b2_ref, w3_ref, b3_ref, b4_ref, out_ref,
               acc_ref, numer_ref, denom_ref):
  t = pl.program_id(0)

  c3 = cnt_ref[...]
  if NPLANE == 1:
    c_t = c3
  else:
    fold = _onehot(BB, BB * NPLANE, lambda r: r, div=NPLANE)
    c_t = _nn(fold, c3)
  w1 = w1t_ref[...]

  s_t = jnp.tanh(_nn(a_ref[...], w1))
  e_t = jnp.exp(s_t)

  @pl.when(t == 0)
  def _():
    acc_ref[...] = jnp.zeros_like(acc_ref)
    numer_ref[...] = jnp.zeros_like(numer_ref)
    denom_ref[...] = jnp.zeros_like(denom_ref)

  denom_ref[...] += _nt(c_t, e_t)
  ohb = _onehot(BB * DA, BB, lambda r: r // DA)
  oha = _onehot(BB * DA, DA, lambda r: r % DA)
  r_t = _nn(ohb, c_t) * _nn(oha, e_t)
  numer_ref[...] += _nt(r_t, w1)

  gt = _tn(w1, w4_ref[...])
  q = gt * pc_ref[...]
  acc_ref[...] += _nn(c_t, q)

  @pl.when(t == NTILES - 1)
  def _():
    numer = numer_ref[...].reshape(BB, DA, H1)
    denom = denom_ref[...]
    emb = numer / denom[:, :, None]
    lz = jnp.sum(emb * wsa_ref[...][0][None, :, None], axis=1) + bsa_ref[0]
    z = jnp.tanh(lz)
    z = jnp.tanh(_nt(z, w2_ref[...]) + b2_ref[...][None, :])
    dz = jnp.tanh(_nt(z, w3_ref[...]) + b3_ref[...][None, :])
    y = _nt(dz, w4_ref[...]) + b4_ref[...][None, :] + acc_ref[...]
    out_ref[...] = jax.nn.sigmoid(y)


def _tc_main(counts2, w1t, pc, w4, a, wsa, bsa, w2, b2, w3, b3, b4):
  grid = (NTILES,)
  full = lambda shape: pl.BlockSpec(shape, lambda t: (0,) * len(shape))
  return pl.pallas_call(
      _main_body,
      grid=grid,
      in_specs=[
          pl.BlockSpec((BB * NPLANE, TILE), lambda t: (0, t)),
          pl.BlockSpec((H1, TILE), lambda t: (0, t)),
          pl.BlockSpec((TILE, D_OUT), lambda t: (t, 0)),
          full((D_OUT, H1)),
          full((DA, H1)),
          full((1, DA)),
          full((1,)),
          full((H, H1)),
          full((H,)),
          full((H1, H)),
          full((H1,)),
          full((D_OUT,)),
      ],
      out_specs=pl.BlockSpec((BB, D_OUT), lambda t: (0, 0)),
      out_shape=jax.ShapeDtypeStruct((BB, D_OUT), jnp.float32),
      scratch_shapes=[
          pltpu.VMEM((BB, D_OUT), jnp.float32),
          pltpu.VMEM((BB * DA, H1), jnp.float32),
          pltpu.VMEM((BB, DA), jnp.float32),
      ],
  )(counts2, w1t, pc, w4, a, wsa, bsa, w2, b2, w3, b3, b4)


def kernel(batch_item_index, place_correlation, W1, W2, b2, W3, b3, W4, b4,
           A, Wsa, bsa):
  zeros_flat = jnp.zeros((NPLANE, D_IN), jnp.float32)
  counts2 = _sc_counts(batch_item_index, zeros_flat)
  return _tc_main(
      counts2,
      W1,
      place_correlation,
      W4,
      A,
      Wsa,
      bsa,
      W2,
      b2,
      W3,
      b3,
      b4,
  )
